# Initial kernel scaffold; baseline (speedup 1.0000x reference)
import jax
import jax.numpy as jnp
from jax.experimental import pallas as pl


def kernel(x, keypoint_y):
    raise NotImplementedError("write your pallas kernel here")


# speedup vs baseline: 2.9380x; 2.9380x over previous
import functools

import jax
import jax.numpy as jnp
from jax import lax
from jax.experimental import pallas as pl
from jax.experimental.pallas import tpu as pltpu
from jax.experimental.pallas import tpu_sc as plsc

NC, NS, L = 2, 16, 16
NW = NC * NS
N_KP = 16
TOTAL = 16384 * 100
CHUNK = TOTAL // NW

def _vgather(vec, idx):
    dn = lax.GatherDimensionNumbers(
        offset_dims=(), collapsed_slice_dims=(0,), start_index_map=(0,)
    )
    return lax.gather(
        vec, idx[:, None], dn, slice_sizes=(1,),
        mode=lax.GatherScatterMode.PROMISE_IN_BOUNDS,
    )


_mesh = plsc.VectorSubcoreMesh(
    core_axis_name="c", subcore_axis_name="s", num_cores=NC, num_subcores=NS
)


@functools.partial(
    pl.kernel,
    out_type=jax.ShapeDtypeStruct((TOTAL,), jnp.float32),
    mesh=_mesh,
    scratch_types=[
        pltpu.VMEM((CHUNK,), jnp.float32),
        pltpu.VMEM((CHUNK,), jnp.float32),
        pltpu.VMEM((N_KP,), jnp.float32),
        pltpu.VMEM((N_KP,), jnp.float32),
        pltpu.VMEM((N_KP,), jnp.float32),
    ],
)
def _calib(x_hbm, kp_hbm, out_hbm, x_v, y_v, kp_v, tab_a, tab_b):
    wid = lax.axis_index("s") * NC + lax.axis_index("c")
    base = wid * CHUNK

    pltpu.sync_copy(kp_hbm, kp_v)
    pltpu.sync_copy(x_hbm.at[pl.ds(base, CHUNK)], x_v)

    lane = lax.iota(jnp.int32, L)
    lane_l = jnp.maximum(lane - 1, 0)
    raw = kp_v[...]
    y_r = 1.0 / (1.0 + jnp.exp(-raw))
    y_l = _vgather(y_r, lane_l)
    x_r = lane.astype(jnp.float32) * (1.0 / 15.0)
    x_l = lane_l.astype(jnp.float32) * (1.0 / 15.0)
    tab_b_vec = (y_r - y_l) / (x_r - x_l + 1e-8)
    tab_a_vec = y_l - x_l * tab_b_vec
    tab_a[...] = tab_a_vec
    tab_b[...] = tab_b_vec

    def body(i, carry):
        off = i * L
        v = x_v[pl.ds(off, L)]
        vc = jnp.minimum(jnp.maximum(v, 0.0), 1.0)
        idx = jnp.minimum((vc * 15.0).astype(jnp.int32) + 1, 15)
        av = _vgather(tab_a[...], idx)
        bv = _vgather(tab_b[...], idx)
        y_v[pl.ds(off, L)] = av + vc * bv
        return carry

    lax.fori_loop(0, CHUNK // L, body, 0)
    pltpu.sync_copy(y_v, out_hbm.at[pl.ds(base, CHUNK)])


def kernel(x, keypoint_y):
    out = _calib(x.reshape(-1), keypoint_y)
    return out.reshape(x.shape)


# speedup vs baseline: 3.4669x; 1.1800x over previous
import functools

import jax
import jax.numpy as jnp
from jax import lax
from jax.experimental import pallas as pl
from jax.experimental.pallas import tpu as pltpu
from jax.experimental.pallas import tpu_sc as plsc

NC, NS, L = 2, 16, 16
NW = NC * NS
N_KP = 16
TOTAL = 16384 * 100
CHUNK = TOTAL // NW

def _vgather(vec, idx):
    dn = lax.GatherDimensionNumbers(
        offset_dims=(), collapsed_slice_dims=(0,), start_index_map=(0,)
    )
    return lax.gather(
        vec, idx[:, None], dn, slice_sizes=(1,),
        mode=lax.GatherScatterMode.PROMISE_IN_BOUNDS,
    )


_mesh = plsc.VectorSubcoreMesh(
    core_axis_name="c", subcore_axis_name="s", num_cores=NC, num_subcores=NS
)


@functools.partial(
    pl.kernel,
    out_type=jax.ShapeDtypeStruct((TOTAL,), jnp.float32),
    mesh=_mesh,
    scratch_types=[
        pltpu.VMEM((CHUNK,), jnp.float32),
        pltpu.VMEM((CHUNK,), jnp.float32),
        pltpu.VMEM((N_KP,), jnp.float32),
    ],
)
def _calib(x_hbm, kp_hbm, out_hbm, x_v, y_v, kp_v):
    wid = lax.axis_index("s") * NC + lax.axis_index("c")
    base = wid * CHUNK

    pltpu.sync_copy(kp_hbm, kp_v)
    pltpu.sync_copy(x_hbm.at[pl.ds(base, CHUNK)], x_v)

    lane = lax.iota(jnp.int32, L)
    lane_l = jnp.maximum(lane - 1, 0)
    raw = kp_v[...]
    y_r = 1.0 / (1.0 + jnp.exp(-raw))
    y_l = _vgather(y_r, lane_l)
    x_r = lane.astype(jnp.float32) * (1.0 / 15.0)
    x_l = lane_l.astype(jnp.float32) * (1.0 / 15.0)
    tab_b_vec = (y_r - y_l) / (x_r - x_l + 1e-8)
    tab_a_vec = y_l - x_l * tab_b_vec

    @plsc.parallel_loop(0, CHUNK, step=L, unroll=8)
    def _body(off):
        v = x_v[pl.ds(off, L)]
        vc = jnp.minimum(jnp.maximum(v, 0.0), 1.0)
        idx = jnp.minimum((vc * 15.0).astype(jnp.int32) + 1, 15)
        av = _vgather(tab_a_vec, idx)
        bv = _vgather(tab_b_vec, idx)
        y_v[pl.ds(off, L)] = av + vc * bv
    pltpu.sync_copy(y_v, out_hbm.at[pl.ds(base, CHUNK)])


def kernel(x, keypoint_y):
    out = _calib(x.reshape(-1), keypoint_y)
    return out.reshape(x.shape)


# speedup vs baseline: 5.3251x; 1.5360x over previous
import functools

import jax
import jax.numpy as jnp
from jax import lax
from jax.experimental import pallas as pl
from jax.experimental.pallas import tpu as pltpu
from jax.experimental.pallas import tpu_sc as plsc

NC, NS, L = 2, 16, 16
NW = NC * NS
N_KP = 16
ROWS, COLS = 16384, 100
ROWS_PER_TILE = ROWS // NW
RCHUNK = 256
COL_OFFS = (0, 16, 32, 48, 64, 80, 84)


def _vgather(vec, idx):
    dn = lax.GatherDimensionNumbers(
        offset_dims=(), collapsed_slice_dims=(0,), start_index_map=(0,)
    )
    return lax.gather(
        vec, idx[:, None], dn, slice_sizes=(1,),
        mode=lax.GatherScatterMode.PROMISE_IN_BOUNDS,
    )


_mesh = plsc.VectorSubcoreMesh(
    core_axis_name="c", subcore_axis_name="s", num_cores=NC, num_subcores=NS
)


@functools.partial(
    pl.kernel,
    out_type=jax.ShapeDtypeStruct((ROWS, COLS), jnp.float32),
    mesh=_mesh,
    compiler_params=pltpu.CompilerParams(use_tc_tiling_on_sc=True),
    scratch_types=[
        pltpu.VMEM((RCHUNK, COLS), jnp.float32),
        pltpu.VMEM((RCHUNK, COLS), jnp.float32),
        pltpu.VMEM((N_KP,), jnp.float32),
    ],
)
def _calib(x_hbm, kp_hbm, out_hbm, x_v, y_v, kp_v):
    wid = lax.axis_index("s") * NC + lax.axis_index("c")
    base_row = wid * ROWS_PER_TILE

    pltpu.sync_copy(kp_hbm, kp_v)

    lane = lax.iota(jnp.int32, L)
    lane_l = jnp.maximum(lane - 1, 0)
    raw = kp_v[...]
    y_r = 1.0 / (1.0 + jnp.exp(-raw))
    y_l = _vgather(y_r, lane_l)
    x_r = lane.astype(jnp.float32) * (1.0 / 15.0)
    x_l = lane_l.astype(jnp.float32) * (1.0 / 15.0)
    tab_b_vec = (y_r - y_l) / (x_r - x_l + 1e-8)
    tab_a_vec = y_l - x_l * tab_b_vec

    for chunk in range(ROWS_PER_TILE // RCHUNK):
        r0 = base_row + chunk * RCHUNK
        pltpu.sync_copy(x_hbm.at[pl.ds(r0, RCHUNK), :], x_v)

        @plsc.parallel_loop(0, RCHUNK, step=1, unroll=2)
        def _body(r):
            for c in COL_OFFS:
                v = x_v[r, pl.ds(c, L)]
                vc = jnp.minimum(jnp.maximum(v, 0.0), 1.0)
                idx = jnp.clip((vc * 15.0).astype(jnp.int32) + 1, 1, 15)
                av = _vgather(tab_a_vec, idx)
                bv = _vgather(tab_b_vec, idx)
                y_v[r, pl.ds(c, L)] = av + vc * bv

        pltpu.sync_copy(y_v, out_hbm.at[pl.ds(r0, RCHUNK), :])


def kernel(x, keypoint_y):
    return _calib(x, keypoint_y)


# speedup vs baseline: 5.6803x; 1.0667x over previous
import functools

import jax
import jax.numpy as jnp
from jax import lax
from jax.experimental import pallas as pl
from jax.experimental.pallas import tpu as pltpu
from jax.experimental.pallas import tpu_sc as plsc

NC, NS, L = 2, 16, 16
NW = NC * NS
N_KP = 16
ROWS, COLS = 16384, 100
ROWS_PER_TILE = ROWS // NW
RCHUNK = 128
NCHUNK = ROWS_PER_TILE // RCHUNK
COL_OFFS = (0, 16, 32, 48, 64, 80, 84)


def _vgather(vec, idx):
    dn = lax.GatherDimensionNumbers(
        offset_dims=(), collapsed_slice_dims=(0,), start_index_map=(0,)
    )
    return lax.gather(
        vec, idx[:, None], dn, slice_sizes=(1,),
        mode=lax.GatherScatterMode.PROMISE_IN_BOUNDS,
    )


_mesh = plsc.VectorSubcoreMesh(
    core_axis_name="c", subcore_axis_name="s", num_cores=NC, num_subcores=NS
)


@functools.partial(
    pl.kernel,
    out_type=jax.ShapeDtypeStruct((ROWS, COLS), jnp.float32),
    mesh=_mesh,
    compiler_params=pltpu.CompilerParams(use_tc_tiling_on_sc=True),
    scratch_types=[
        pltpu.VMEM((RCHUNK, COLS), jnp.float32),
        pltpu.VMEM((RCHUNK, COLS), jnp.float32),
        pltpu.VMEM((RCHUNK, COLS), jnp.float32),
        pltpu.VMEM((RCHUNK, COLS), jnp.float32),
        pltpu.VMEM((N_KP,), jnp.float32),
        pltpu.SemaphoreType.DMA,
        pltpu.SemaphoreType.DMA,
        pltpu.SemaphoreType.DMA,
        pltpu.SemaphoreType.DMA,
    ],
)
def _calib(x_hbm, kp_hbm, out_hbm,
           x_v0, x_v1, y_v0, y_v1, kp_v,
           in_sem0, in_sem1, out_sem0, out_sem1):
    x_bufs = (x_v0, x_v1)
    y_bufs = (y_v0, y_v1)
    in_sems = (in_sem0, in_sem1)
    out_sems = (out_sem0, out_sem1)
    wid = lax.axis_index("s") * NC + lax.axis_index("c")
    base_row = wid * ROWS_PER_TILE

    pltpu.sync_copy(kp_hbm, kp_v)

    lane = lax.iota(jnp.int32, L)
    lane_l = jnp.maximum(lane - 1, 0)
    raw = kp_v[...]
    y_r = 1.0 / (1.0 + jnp.exp(-raw))
    y_l = _vgather(y_r, lane_l)
    x_r = lane.astype(jnp.float32) * (1.0 / 15.0)
    x_l = lane_l.astype(jnp.float32) * (1.0 / 15.0)
    tab_b_vec = (y_r - y_l) / (x_r - x_l + 1e-8)
    tab_a_vec = y_l - x_l * tab_b_vec

    in_dma = [None] * NCHUNK
    out_dma = [None] * NCHUNK
    in_dma[0] = pltpu.async_copy(
        x_hbm.at[pl.ds(base_row, RCHUNK), :], x_bufs[0], in_sems[0])
    for k in range(NCHUNK):
        cur = k % 2
        r0 = base_row + k * RCHUNK
        in_dma[k].wait()
        if k + 1 < NCHUNK:
            in_dma[k + 1] = pltpu.async_copy(
                x_hbm.at[pl.ds(r0 + RCHUNK, RCHUNK), :],
                x_bufs[1 - cur], in_sems[1 - cur])
        if k >= 2:
            out_dma[k - 2].wait()
        x_v = x_bufs[cur]
        y_v = y_bufs[cur]

        @plsc.parallel_loop(0, RCHUNK, step=1, unroll=2)
        def _body(r):
            for c in COL_OFFS:
                v = x_v[r, pl.ds(c, L)]
                vc = jnp.minimum(jnp.maximum(v, 0.0), 1.0)
                idx = jnp.clip((vc * 15.0).astype(jnp.int32) + 1, 1, 15)
                av = _vgather(tab_a_vec, idx)
                bv = _vgather(tab_b_vec, idx)
                y_v[r, pl.ds(c, L)] = av + vc * bv

        out_dma[k] = pltpu.async_copy(
            y_v, out_hbm.at[pl.ds(r0, RCHUNK), :], out_sems[cur])
    out_dma[NCHUNK - 2].wait()
    out_dma[NCHUNK - 1].wait()


def kernel(x, keypoint_y):
    return _calib(x, keypoint_y)


# speedup vs baseline: 5.7977x; 1.0207x over previous
import functools

import jax
import jax.numpy as jnp
from jax import lax
from jax.experimental import pallas as pl
from jax.experimental.pallas import tpu as pltpu
from jax.experimental.pallas import tpu_sc as plsc

NC, NS, L = 2, 16, 16
NW = NC * NS
N_KP = 16
ROWS, COLS = 16384, 100
ROWS_PER_TILE = ROWS // NW
RCHUNK = 128
NCHUNK = ROWS_PER_TILE // RCHUNK
COL_OFFS = (0, 16, 32, 48, 64, 80, 84)


def _vgather(vec, idx):
    dn = lax.GatherDimensionNumbers(
        offset_dims=(), collapsed_slice_dims=(0,), start_index_map=(0,)
    )
    return lax.gather(
        vec, idx[:, None], dn, slice_sizes=(1,),
        mode=lax.GatherScatterMode.PROMISE_IN_BOUNDS,
    )


_mesh = plsc.VectorSubcoreMesh(
    core_axis_name="c", subcore_axis_name="s", num_cores=NC, num_subcores=NS
)


@functools.partial(
    pl.kernel,
    out_type=jax.ShapeDtypeStruct((ROWS, COLS), jnp.float32),
    mesh=_mesh,
    compiler_params=pltpu.CompilerParams(use_tc_tiling_on_sc=True),
    scratch_types=[
        pltpu.VMEM((RCHUNK, COLS), jnp.float32),
        pltpu.VMEM((RCHUNK, COLS), jnp.float32),
        pltpu.VMEM((RCHUNK, COLS), jnp.float32),
        pltpu.VMEM((RCHUNK, COLS), jnp.float32),
        pltpu.VMEM((N_KP,), jnp.float32),
        pltpu.SemaphoreType.DMA,
        pltpu.SemaphoreType.DMA,
        pltpu.SemaphoreType.DMA,
        pltpu.SemaphoreType.DMA,
    ],
)
def _calib(x_hbm, kp_hbm, out_hbm,
           x_v0, x_v1, y_v0, y_v1, kp_v,
           in_sem0, in_sem1, out_sem0, out_sem1):
    x_bufs = (x_v0, x_v1)
    y_bufs = (y_v0, y_v1)
    in_sems = (in_sem0, in_sem1)
    out_sems = (out_sem0, out_sem1)
    wid = lax.axis_index("s") * NC + lax.axis_index("c")
    base_row = wid * ROWS_PER_TILE

    pltpu.sync_copy(kp_hbm, kp_v)

    lane = lax.iota(jnp.int32, L)
    lane_l = jnp.maximum(lane - 1, 0)
    raw = kp_v[...]
    y_r = 1.0 / (1.0 + jnp.exp(-raw))
    y_l = _vgather(y_r, lane_l)
    x_r = lane.astype(jnp.float32) * (1.0 / 15.0)
    x_l = lane_l.astype(jnp.float32) * (1.0 / 15.0)
    tab_b_seg = (y_r - y_l) / (x_r - x_l + 1e-8)
    tab_a_seg = y_l - x_l * tab_b_seg
    tab_b_vec = _vgather(tab_b_seg, jnp.minimum(lane + 1, 15))
    tab_a_vec = _vgather(tab_a_seg, jnp.minimum(lane + 1, 15))

    in_dma = [None] * NCHUNK
    out_dma = [None] * NCHUNK
    in_dma[0] = pltpu.async_copy(
        x_hbm.at[pl.ds(base_row, RCHUNK), :], x_bufs[0], in_sems[0])
    for k in range(NCHUNK):
        cur = k % 2
        r0 = base_row + k * RCHUNK
        in_dma[k].wait()
        if k + 1 < NCHUNK:
            in_dma[k + 1] = pltpu.async_copy(
                x_hbm.at[pl.ds(r0 + RCHUNK, RCHUNK), :],
                x_bufs[1 - cur], in_sems[1 - cur])
        if k >= 2:
            out_dma[k - 2].wait()
        x_v = x_bufs[cur]
        y_v = y_bufs[cur]

        @plsc.parallel_loop(0, RCHUNK, step=1, unroll=4)
        def _body(r):
            for c in COL_OFFS:
                v = x_v[r, pl.ds(c, L)]
                vc = jnp.minimum(jnp.maximum(v, 0.0), 1.0)
                idx = jnp.minimum((vc * 15.0).astype(jnp.int32), 14)
                av = _vgather(tab_a_vec, idx)
                bv = _vgather(tab_b_vec, idx)
                y_v[r, pl.ds(c, L)] = av + vc * bv

        out_dma[k] = pltpu.async_copy(
            y_v, out_hbm.at[pl.ds(r0, RCHUNK), :], out_sems[cur])
    out_dma[NCHUNK - 2].wait()
    out_dma[NCHUNK - 1].wait()


def kernel(x, keypoint_y):
    return _calib(x, keypoint_y)


# speedup vs baseline: 5.8113x; 1.0023x over previous
import functools

import jax
import jax.numpy as jnp
from jax import lax
from jax.experimental import pallas as pl
from jax.experimental.pallas import tpu as pltpu
from jax.experimental.pallas import tpu_sc as plsc

NC, NS, L = 2, 16, 16
NW = NC * NS
N_KP = 16
ROWS, COLS = 16384, 100
ROWS_PER_TILE = ROWS // NW
RCHUNK = 128
NCHUNK = ROWS_PER_TILE // RCHUNK
COL_OFFS = (0, 16, 32, 48, 64, 80, 84)


def _vgather(vec, idx):
    dn = lax.GatherDimensionNumbers(
        offset_dims=(), collapsed_slice_dims=(0,), start_index_map=(0,)
    )
    return lax.gather(
        vec, idx[:, None], dn, slice_sizes=(1,),
        mode=lax.GatherScatterMode.PROMISE_IN_BOUNDS,
    )


_mesh = plsc.VectorSubcoreMesh(
    core_axis_name="c", subcore_axis_name="s", num_cores=NC, num_subcores=NS
)


@functools.partial(
    pl.kernel,
    out_type=jax.ShapeDtypeStruct((ROWS, COLS), jnp.float32),
    mesh=_mesh,
    compiler_params=pltpu.CompilerParams(
        use_tc_tiling_on_sc=True, has_side_effects=True
    ),
    scratch_types=[
        pltpu.VMEM((RCHUNK, COLS), jnp.float32),
        pltpu.VMEM((RCHUNK, COLS), jnp.float32),
        pltpu.VMEM((RCHUNK, COLS), jnp.float32),
        pltpu.VMEM((RCHUNK, COLS), jnp.float32),
        pltpu.VMEM((N_KP,), jnp.float32),
        pltpu.SemaphoreType.DMA,
        pltpu.SemaphoreType.DMA,
        pltpu.SemaphoreType.DMA,
        pltpu.SemaphoreType.DMA,
    ],
)
def _calib(x_hbm, kp_hbm, out_hbm,
           x_v0, x_v1, y_v0, y_v1, kp_v,
           in_sem0, in_sem1, out_sem0, out_sem1):
    x_bufs = (x_v0, x_v1)
    y_bufs = (y_v0, y_v1)
    in_sems = (in_sem0, in_sem1)
    out_sems = (out_sem0, out_sem1)
    wid = lax.axis_index("s") * NC + lax.axis_index("c")
    base_row = wid * ROWS_PER_TILE

    pltpu.sync_copy(kp_hbm, kp_v)

    lane = lax.iota(jnp.int32, L)
    lane_l = jnp.maximum(lane - 1, 0)
    raw = kp_v[...]
    y_r = 1.0 / (1.0 + jnp.exp(-raw))
    y_l = _vgather(y_r, lane_l)
    x_r = lane.astype(jnp.float32) * (1.0 / 15.0)
    x_l = lane_l.astype(jnp.float32) * (1.0 / 15.0)
    tab_b_seg = (y_r - y_l) / (x_r - x_l + 1e-8)
    tab_a_seg = y_l - x_l * tab_b_seg
    tab_b_vec = _vgather(tab_b_seg, jnp.minimum(lane + 1, 15))
    tab_a_vec = _vgather(tab_a_seg, jnp.minimum(lane + 1, 15))

    in_dma = [None] * NCHUNK
    out_dma = [None] * NCHUNK
    in_dma[0] = pltpu.async_copy(
        x_hbm.at[pl.ds(base_row, RCHUNK), :], x_bufs[0], in_sems[0])
    for k in range(NCHUNK):
        cur = k % 2
        r0 = base_row + k * RCHUNK
        in_dma[k].wait()
        if k + 1 < NCHUNK:
            in_dma[k + 1] = pltpu.async_copy(
                x_hbm.at[pl.ds(r0 + RCHUNK, RCHUNK), :],
                x_bufs[1 - cur], in_sems[1 - cur])
        if k >= 2:
            out_dma[k - 2].wait()
        x_v = x_bufs[cur]
        y_v = y_bufs[cur]

        @plsc.parallel_loop(0, RCHUNK, step=1, unroll=4)
        def _body(r):
            for c in COL_OFFS:
                v = x_v[r, pl.ds(c, L)]
                vc = jnp.minimum(jnp.maximum(v, 0.0), 1.0)
                idx = jnp.minimum((vc * 15.0).astype(jnp.int32), 14)
                av = _vgather(tab_a_vec, idx)
                bv = _vgather(tab_b_vec, idx)
                y_v[r, pl.ds(c, L)] = av + vc * bv

        out_dma[k] = pltpu.async_copy(
            y_v, out_hbm.at[pl.ds(r0, RCHUNK), :], out_sems[cur])
    out_dma[NCHUNK - 2].wait()
    out_dma[NCHUNK - 1].wait()


def kernel(x, keypoint_y):
    return _calib(x, keypoint_y)
